# baseline (speedup 1.0000x reference)
import functools

import jax
import jax.numpy as jnp
from jax.experimental import pallas as pl
from jax.experimental.pallas import tpu as pltpu

_NIN = 2
_NOUT = 2


def _pipelined_linear(nchunks, ch, h_hbm, wt_ref, b_ref, o_hbm,
                      inbuf, outbuf, insem, outsem):
    def in_copy(i):
        slot = i % _NIN
        return pltpu.make_async_copy(
            h_hbm.at[pl.ds(i * ch, ch), :], inbuf.at[slot], insem.at[slot]
        )

    def out_copy(i):
        slot = i % _NOUT
        return pltpu.make_async_copy(
            outbuf.at[slot], o_hbm.at[pl.ds(i * ch, ch), :], outsem.at[slot]
        )

    for i in range(min(_NIN, nchunks)):
        in_copy(i).start()
    for i in range(nchunks):
        in_copy(i).wait()
        if i >= _NOUT:
            out_copy(i - _NOUT).wait()
        islot, oslot = i % _NIN, i % _NOUT
        acc = jnp.dot(inbuf[islot], wt_ref[:, :], preferred_element_type=jnp.float32)
        outbuf[oslot] = acc + b_ref[:, :]
        out_copy(i).start(priority=1)
        if i + _NIN < nchunks:
            in_copy(i + _NIN).start()
    for i in range(max(0, nchunks - _NOUT), nchunks):
        out_copy(i).wait()


def kernel(h, edge_index, edge_sh, edge_radial, n_atoms, W, b):
    n, d = h.shape
    ch = 5000
    nchunks = n // ch if (n % ch == 0) else 1
    if n % ch != 0:
        ch = n
    wt = W.T
    b2 = b.reshape(1, d)
    return pl.pallas_call(
        functools.partial(_pipelined_linear, nchunks, ch),
        in_specs=[
            pl.BlockSpec(memory_space=pl.ANY),
            pl.BlockSpec((d, d), lambda: (0, 0)),
            pl.BlockSpec((1, d), lambda: (0, 0)),
        ],
        out_specs=pl.BlockSpec(memory_space=pl.ANY),
        out_shape=jax.ShapeDtypeStruct((n, d), jnp.float32),
        scratch_shapes=[
            pltpu.VMEM((_NIN, ch, d), jnp.float32),
            pltpu.VMEM((_NOUT, ch, d), jnp.float32),
            pltpu.SemaphoreType.DMA((_NIN,)),
            pltpu.SemaphoreType.DMA((_NOUT,)),
        ],
    )(h, wt, b2)


# speedup vs baseline: 1.3059x; 1.3059x over previous
import functools

import jax
import jax.numpy as jnp
from jax.experimental import pallas as pl
from jax.experimental.pallas import tpu as pltpu


def _linear_kernel(nchunks, ch, h_ref, wt_ref, b_ref, o_hbm, outbuf, outsem):
    for i in range(nchunks):
        rows = pl.ds(i * ch, ch)
        acc = jnp.dot(h_ref[rows, :], wt_ref[:, :],
                      preferred_element_type=jnp.float32)
        outbuf[rows, :] = acc + b_ref[:, :]
        pltpu.make_async_copy(
            outbuf.at[rows, :], o_hbm.at[rows, :], outsem.at[i]
        ).start(priority=1)
    for i in range(nchunks):
        pltpu.make_async_copy(
            outbuf.at[pl.ds(i * ch, ch), :],
            o_hbm.at[pl.ds(i * ch, ch), :],
            outsem.at[i],
        ).wait()


def kernel(h, edge_index, edge_sh, edge_radial, n_atoms, W, b):
    n, d = h.shape
    ch = 2000
    nchunks = n // ch if (n % ch == 0) else 1
    if n % ch != 0:
        ch = n
    wt = W.T
    b2 = b.reshape(1, d)

    return pl.pallas_call(
        functools.partial(_linear_kernel, nchunks, ch),
        in_specs=[
            pl.BlockSpec(memory_space=pltpu.VMEM),
            pl.BlockSpec(memory_space=pltpu.VMEM),
            pl.BlockSpec(memory_space=pltpu.VMEM),
        ],
        out_specs=pl.BlockSpec(memory_space=pl.ANY),
        out_shape=jax.ShapeDtypeStruct((n, d), jnp.float32),
        scratch_shapes=[
            pltpu.VMEM((n, d), jnp.float32),
            pltpu.SemaphoreType.DMA((nchunks,)),
        ],
    )(h, wt, b2)
